# baseline (speedup 1.0000x reference)
import functools

import jax
import jax.numpy as jnp
from jax import lax
from jax.experimental import pallas as pl
from jax.experimental.pallas import tpu as pltpu
from jax.experimental.pallas import tpu_sc as plsc

NUM_CORES = 2
NUM_SUBCORES = 16
NUM_WORKERS = NUM_CORES * NUM_SUBCORES
LANES = 16


def _build_sc_embed(batch, maxlen, embed):
  rows_total = batch * maxlen
  assert batch % NUM_WORKERS == 0
  seq_per_w = batch // NUM_WORKERS
  rows_per_w = seq_per_w * maxlen
  assert maxlen == 200 and embed % LANES == 0

  mesh = plsc.VectorSubcoreMesh(core_axis_name="c", subcore_axis_name="s")

  @functools.partial(
      pl.kernel,
      mesh=mesh,
      compiler_params=pltpu.CompilerParams(use_tc_tiling_on_sc=False),
      out_type=jax.ShapeDtypeStruct((rows_total, embed), jnp.float32),
      scratch_types=[
          pltpu.VMEM((rows_per_w,), jnp.int32),
          pltpu.VMEM((maxlen, embed), jnp.float32),
          pltpu.VMEM((maxlen, embed), jnp.float32),
          pltpu.SemaphoreType.DMA,
          pltpu.SemaphoreType.DMA,
      ],
  )
  def k(table_hbm, idx_hbm, pos_hbm, out_hbm, idx_v, pos_v, buf, sem_in,
        sem_out):
    wid = lax.axis_index("s") * NUM_CORES + lax.axis_index("c")
    base = pl.multiple_of(wid * rows_per_w, 8)
    pltpu.sync_copy(pos_hbm, pos_v)
    pltpu.sync_copy(idx_hbm.at[pl.ds(base, rows_per_w)], idx_v)

    @pl.loop(0, seq_per_w)
    def _(c):
      off = pl.multiple_of(c * maxlen, 8)
      g0 = pltpu.async_copy(
          table_hbm.at[idx_v.at[pl.ds(off, 128)]],
          buf.at[pl.ds(0, 128)], sem_in)
      g1 = pltpu.async_copy(
          table_hbm.at[idx_v.at[pl.ds(off + 128, maxlen - 128)]],
          buf.at[pl.ds(128, maxlen - 128)], sem_in)
      g0.wait()
      g1.wait()

      @pl.loop(0, maxlen)
      def _(r):
        for c16 in range(embed // LANES):
          sl = (r, pl.ds(c16 * LANES, LANES))
          buf.at[sl][...] = buf.at[sl][...] + pos_v.at[sl][...]

      pltpu.async_copy(buf, out_hbm.at[pl.ds(base + off, maxlen)],
                       sem_out).wait()

  return k


@jax.jit
def kernel(x, token_table, pos_table):
  batch, maxlen = x.shape
  embed = token_table.shape[1]
  x_flat = x.reshape(-1).astype(jnp.int32)
  k = _build_sc_embed(batch, maxlen, embed)
  out = k(token_table, x_flat, pos_table)
  return out.reshape(batch, maxlen, embed)


# speedup vs baseline: 1.0553x; 1.0553x over previous
import functools

import jax
import jax.numpy as jnp
from jax import lax
from jax.experimental import pallas as pl
from jax.experimental.pallas import tpu as pltpu
from jax.experimental.pallas import tpu_sc as plsc

NUM_CORES = 2
NUM_SUBCORES = 16
NUM_WORKERS = NUM_CORES * NUM_SUBCORES
LANES = 16
NBUF = 4


def _build_sc_embed(batch, maxlen, embed):
  rows_total = batch * maxlen
  assert batch % NUM_WORKERS == 0
  seq_per_w = batch // NUM_WORKERS
  rows_per_w = seq_per_w * maxlen
  assert maxlen == 200 and embed % LANES == 0 and seq_per_w % NBUF == 0

  mesh = plsc.VectorSubcoreMesh(core_axis_name="c", subcore_axis_name="s")

  row_buf = pltpu.VMEM((maxlen, embed), jnp.float32)

  @functools.partial(
      pl.kernel,
      mesh=mesh,
      compiler_params=pltpu.CompilerParams(use_tc_tiling_on_sc=False),
      out_type=jax.ShapeDtypeStruct((rows_total, embed), jnp.float32),
      scratch_types=[
          pltpu.VMEM((rows_per_w,), jnp.int32),
          pltpu.VMEM((maxlen, embed), jnp.float32),
          [row_buf] * NBUF,
          [pltpu.SemaphoreType.DMA] * NBUF,
          [pltpu.SemaphoreType.DMA] * NBUF,
      ],
  )
  def k(table_hbm, idx_hbm, pos_hbm, out_hbm, idx_v, pos_v, bufs, sems_in,
        sems_out):
    wid = lax.axis_index("s") * NUM_CORES + lax.axis_index("c")
    base = pl.multiple_of(wid * rows_per_w, 8)
    pltpu.sync_copy(pos_hbm, pos_v)
    pltpu.sync_copy(idx_hbm.at[pl.ds(base, rows_per_w)], idx_v)

    def issue_gather(c, b):
      off = c * maxlen
      g0 = pltpu.async_copy(
          table_hbm.at[idx_v.at[pl.ds(off, 128)]],
          bufs[b].at[pl.ds(0, 128)], sems_in[b])
      g1 = pltpu.async_copy(
          table_hbm.at[idx_v.at[pl.ds(off + 128, maxlen - 128)]],
          bufs[b].at[pl.ds(128, maxlen - 128)], sems_in[b])
      return (g0, g1)

    def issue_out(c, b):
      return pltpu.async_copy(
          bufs[b], out_hbm.at[pl.ds(base + c * maxlen, maxlen)], sems_out[b])

    gather_h = [issue_gather(c, c) for c in range(NBUF)]
    out_h = [None] * NBUF

    for c in range(seq_per_w):
      b = c % NBUF
      for h in gather_h[b]:
        h.wait()

      @plsc.parallel_loop(0, maxlen, unroll=4)
      def _(r):
        for c16 in range(embed // LANES):
          sl = (r, pl.ds(c16 * LANES, LANES))
          plsc.addupdate(bufs[b].at[sl], pos_v.at[sl][...])

      out_h[b] = issue_out(c, b)

      nc = c + NBUF - 1
      if NBUF <= nc < seq_per_w:
        nb = nc % NBUF
        out_h[nb].wait()
        gather_h[nb] = issue_gather(nc, nb)

    for b in range(NBUF):
      out_h[(seq_per_w - NBUF + b) % NBUF].wait()

  return k


@jax.jit
def kernel(x, token_table, pos_table):
  batch, maxlen = x.shape
  embed = token_table.shape[1]
  x_flat = x.reshape(-1).astype(jnp.int32)
  k = _build_sc_embed(batch, maxlen, embed)
  out = k(token_table, x_flat, pos_table)
  return out.reshape(batch, maxlen, embed)
